# Initial kernel scaffold; baseline (speedup 1.0000x reference)
import jax
import jax.numpy as jnp
from jax.experimental import pallas as pl


def kernel(tgt_id, node_feat, x, edge_index, W_mlp, b_mlp, bn_gamma, bn_beta, W_self1, W_neigh1, b1, W_self2, W_neigh2, b2, W_pred, b_pred):
    raise NotImplementedError("write your pallas kernel here")


# speedup vs baseline: 3.3730x; 3.3730x over previous
import dataclasses
import functools

import jax
import jax.numpy as jnp
from jax import lax
from jax.experimental import pallas as pl
from jax.experimental.pallas import tpu as pltpu
from jax.experimental.pallas import tpu_sc as plsc

N = 10000
E = 320000
HID = 128
B = 1024

NP = 10112
ROWS_PER_TILE = NP // 16
CHUNK = 128
EP = 327680
NCHUNKS = EP // CHUNK
CHUNKS_PER_WORKER = NCHUNKS // 32
IDX_BATCH = 16
SLABS = (128, 128, 128, 128, ROWS_PER_TILE - 4 * 128)


def _leaky(v):
    return jnp.where(v >= 0, v, 0.1 * v)


def _mlp_bn_body(nf, w, b, g, be, out):
    h = jnp.dot(nf[...], w[...], preferred_element_type=jnp.float32) + b[...]
    h = _leaky(h)
    mu = jnp.mean(h, axis=0, keepdims=True)
    var = jnp.mean((h - mu) ** 2, axis=0, keepdims=True)
    out[...] = (h - mu) * lax.rsqrt(var + 1e-5) * g[...] + be[...]


def _sage_body(hin, accp, deg_t, ws, wn, b, out):
    acc = accp[0] + accp[1]
    deg = jnp.sum(deg_t[...], axis=1, keepdims=True)
    inv = 1.0 / jnp.maximum(deg, 1.0)
    aggm = (acc * inv)[:N]
    out[...] = _leaky(
        jnp.dot(hin[...], ws[...], preferred_element_type=jnp.float32)
        + jnp.dot(aggm, wn[...], preferred_element_type=jnp.float32)
        + b[...])


def _pred_body(embed, xg, wp, bp, xp_out, loss_out):
    e = _leaky(embed[...])
    xp = jnp.dot(e, wp[...], preferred_element_type=jnp.float32) + bp[...]
    xp_out[...] = xp
    d = xp - xg[...]
    loss_out[...] = jnp.mean(d * d).reshape(1, 1)


def _tc_call(body, out_shapes):
    return pl.pallas_call(body, out_shape=out_shapes)


def _agg_body(h_hbm, src_hbm, dst_hbm, out_hbm, acc_sh, idx_s, idx_d, rows, sem):
    cid = lax.axis_index("c")
    sid = lax.axis_index("s")
    rbase = sid * ROWS_PER_TILE

    @pl.loop(0, CHUNK)
    def _(r):
        for k in range(HID // 16):
            rows[r, pl.ds(k * 16, 16)] = jnp.zeros((16,), jnp.float32)

    off = 0
    for sz in SLABS:
        pltpu.sync_copy(rows.at[pl.ds(0, sz)],
                        acc_sh.at[pl.ds(rbase + off, sz)])
        off += sz

    plsc.subcore_barrier()

    cbase = (cid * 16 + sid) * CHUNKS_PER_WORKER

    @pl.loop(0, CHUNKS_PER_WORKER // IDX_BATCH)
    def _(g):
        pltpu.sync_copy(src_hbm.at[pl.ds(cbase + g * IDX_BATCH, IDX_BATCH)], idx_s)
        pltpu.sync_copy(dst_hbm.at[pl.ds(cbase + g * IDX_BATCH, IDX_BATCH)], idx_d)

        @pl.loop(0, IDX_BATCH)
        def _(j):
            pltpu.async_copy(h_hbm.at[idx_s.at[j]], rows, sem).wait()
            pltpu.sync_copy(rows, acc_sh.at[idx_d.at[j]], add=True)

    plsc.subcore_barrier()

    off = 0
    for sz in SLABS:
        pltpu.sync_copy(acc_sh.at[pl.ds(rbase + off, sz)],
                        rows.at[pl.ds(0, sz)])
        pltpu.sync_copy(rows.at[pl.ds(0, sz)],
                        out_hbm.at[cid, pl.ds(rbase + off, sz)])
        off += sz


def _sc_aggregate(h, src2d, dst2d):
    mesh = plsc.VectorSubcoreMesh(core_axis_name="c", subcore_axis_name="s")
    call = pl.kernel(
        _agg_body,
        out_type=jax.ShapeDtypeStruct((2, NP, HID), jnp.float32),
        mesh=mesh,
        scratch_types=[
            pltpu.VMEM_SHARED((NP, HID), jnp.float32),
            pltpu.VMEM((IDX_BATCH, CHUNK), jnp.int32),
            pltpu.VMEM((IDX_BATCH, CHUNK), jnp.int32),
            pltpu.VMEM((CHUNK, HID), jnp.float32),
            pltpu.SemaphoreType.DMA,
        ],
    )
    return call(h, src2d, dst2d)


def _deg_body(dst_hbm, out_hbm, ldeg, idx_d):
    cid = lax.axis_index("c")
    sid = lax.axis_index("s")
    wid = cid * 16 + sid

    @pl.loop(0, NP // 16)
    def _(i):
        ldeg[pl.ds(i * 16, 16)] = jnp.zeros((16,), jnp.float32)

    cbase = wid * CHUNKS_PER_WORKER
    ones16 = jnp.ones((16,), jnp.float32)

    @pl.loop(0, CHUNKS_PER_WORKER // IDX_BATCH)
    def _(g):
        pltpu.sync_copy(dst_hbm.at[pl.ds(cbase + g * IDX_BATCH, IDX_BATCH)], idx_d)

        @pl.loop(0, IDX_BATCH)
        def _(j):
            for k in range(CHUNK // 16):
                iv = idx_d[j, pl.ds(k * 16, 16)]
                plsc.addupdate_scatter(ldeg, [iv], ones16)

    pltpu.sync_copy(ldeg, out_hbm.at[pl.ds(wid * NP, NP)])


def _no_layout_params():
    cp = pltpu.CompilerParams()
    if "needs_layout_passes" in pltpu.CompilerParams.__dataclass_fields__:
        cp = dataclasses.replace(cp, needs_layout_passes=False)
    return cp


def _sc_degree(dst2d):
    mesh = plsc.VectorSubcoreMesh(core_axis_name="c", subcore_axis_name="s")
    call = pl.kernel(
        _deg_body,
        out_type=jax.ShapeDtypeStruct((32 * NP,), jnp.float32),
        mesh=mesh,
        compiler_params=_no_layout_params(),
        scratch_types=[
            pltpu.VMEM((NP,), jnp.float32),
            pltpu.VMEM((IDX_BATCH, CHUNK), jnp.int32),
        ],
    )
    return call(dst2d)


def _tgt_gather_body(h2_hbm, x_hbm, tgt_hbm, emb_hbm, xg_hbm, tidx, buf, sem):
    cid = lax.axis_index("c")
    sid = lax.axis_index("s")
    wid = sid * 2 + cid
    pltpu.sync_copy(tgt_hbm, tidx)
    pltpu.async_copy(h2_hbm.at[tidx.at[wid]], buf, sem).wait()
    pltpu.sync_copy(buf, emb_hbm.at[pl.ds(wid * 32, 32)])
    pltpu.async_copy(x_hbm.at[tidx.at[wid]], buf, sem).wait()
    pltpu.sync_copy(buf, xg_hbm.at[pl.ds(wid * 32, 32)])


def _sc_tgt_gather(h2, x, tgt2d):
    mesh = plsc.VectorSubcoreMesh(core_axis_name="c", subcore_axis_name="s")
    call = pl.kernel(
        _tgt_gather_body,
        out_type=(jax.ShapeDtypeStruct((B, HID), jnp.float32),
                  jax.ShapeDtypeStruct((B, HID), jnp.float32)),
        mesh=mesh,
        scratch_types=[
            pltpu.VMEM((32, 32), jnp.int32),
            pltpu.VMEM((32, HID), jnp.float32),
            pltpu.SemaphoreType.DMA,
        ],
    )
    return call(h2, x, tgt2d)


def kernel(tgt_id, node_feat, x, edge_index, W_mlp, b_mlp, bn_gamma, bn_beta,
           W_self1, W_neigh1, b1, W_self2, W_neigh2, b2, W_pred, b_pred):
    f32 = jnp.float32
    tgt2d = tgt_id.astype(jnp.int32).reshape(32, 32)
    src = edge_index[0].astype(jnp.int32)
    dst = edge_index[1].astype(jnp.int32)
    src2d = jnp.concatenate([src, jnp.zeros((EP - E,), jnp.int32)]).reshape(NCHUNKS, CHUNK)
    dst2d = jnp.concatenate([dst, jnp.full((EP - E,), N, jnp.int32)]).reshape(NCHUNKS, CHUNK)

    b_mlp2 = b_mlp.reshape(1, HID)
    g2 = bn_gamma.reshape(1, HID)
    be2 = bn_beta.reshape(1, HID)
    b1_2 = b1.reshape(1, HID)
    b2_2 = b2.reshape(1, HID)
    bp2 = b_pred.reshape(1, HID)

    deg_t = _sc_degree(dst2d).reshape(32, NP).T

    h0 = _tc_call(_mlp_bn_body, jax.ShapeDtypeStruct((N, HID), f32))(
        node_feat, W_mlp, b_mlp2, g2, be2)

    acc1 = _sc_aggregate(h0, src2d, dst2d)
    h1 = _tc_call(_sage_body, jax.ShapeDtypeStruct((N, HID), f32))(
        h0, acc1, deg_t, W_self1, W_neigh1, b1_2)

    acc2 = _sc_aggregate(h1, src2d, dst2d)
    h2 = _tc_call(_sage_body, jax.ShapeDtypeStruct((N, HID), f32))(
        h1, acc2, deg_t, W_self2, W_neigh2, b2_2)

    embed, xg = _sc_tgt_gather(h2, x, tgt2d)

    x_prime, loss_arr = _tc_call(
        _pred_body,
        (jax.ShapeDtypeStruct((B, HID), f32),
         jax.ShapeDtypeStruct((1, 1), f32)))(embed, xg, W_pred, bp2)

    return (loss_arr[0, 0], x_prime, embed)


# speedup vs baseline: 8.0841x; 2.3967x over previous
import dataclasses
import functools

import jax
import jax.numpy as jnp
from jax import lax
from jax.experimental import pallas as pl
from jax.experimental.pallas import tpu as pltpu
from jax.experimental.pallas import tpu_sc as plsc

N = 10000
E = 320000
HID = 128
B = 1024

NP = 10112
ROWS_PER_TILE = NP // 16
CHUNK = 128
EP = 327680
NCHUNKS = EP // CHUNK
CHUNKS_PER_WORKER = NCHUNKS // 32
IDX_BATCH = 16
SLABS = (128, 128, 128, 128, ROWS_PER_TILE - 4 * 128)


def _leaky(v):
    return jnp.where(v >= 0, v, 0.1 * v)


def _mlp_bn_body(nf, w, b, g, be, out):
    h = jnp.dot(nf[...], w[...], preferred_element_type=jnp.float32) + b[...]
    h = _leaky(h)
    mu = jnp.mean(h, axis=0, keepdims=True)
    var = jnp.mean((h - mu) ** 2, axis=0, keepdims=True)
    out[...] = (h - mu) * lax.rsqrt(var + 1e-5) * g[...] + be[...]


def _sage_body(hin, accp, deg_t, ws, wn, b, out):
    acc = accp[0] + accp[1]
    deg = jnp.sum(deg_t[...], axis=1, keepdims=True)
    inv = 1.0 / jnp.maximum(deg, 1.0)
    aggm = (acc * inv)[:N]
    out[...] = _leaky(
        jnp.dot(hin[...], ws[...], preferred_element_type=jnp.float32)
        + jnp.dot(aggm, wn[...], preferred_element_type=jnp.float32)
        + b[...])


def _pred_body(embed, xg, wp, bp, xp_out, loss_out):
    e = _leaky(embed[...])
    xp = jnp.dot(e, wp[...], preferred_element_type=jnp.float32) + bp[...]
    xp_out[...] = xp
    d = xp - xg[...]
    loss_out[...] = jnp.mean(d * d).reshape(1, 1)


def _tc_call(body, out_shapes):
    return pl.pallas_call(body, out_shape=out_shapes)


def _agg_body(h_hbm, src_hbm, dst_hbm, out_hbm, acc_sh, idx_s, idx_d, rows, sem):
    cid = lax.axis_index("c")
    sid = lax.axis_index("s")
    rbase = sid * ROWS_PER_TILE

    @pl.loop(0, CHUNK)
    def _(r):
        for k in range(HID // 16):
            rows[r, pl.ds(k * 16, 16)] = jnp.zeros((16,), jnp.float32)

    off = 0
    for sz in SLABS:
        pltpu.sync_copy(rows.at[pl.ds(0, sz)],
                        acc_sh.at[pl.ds(rbase + off, sz)])
        off += sz

    plsc.subcore_barrier()

    cbase = (cid * 16 + sid) * CHUNKS_PER_WORKER

    @pl.loop(0, CHUNKS_PER_WORKER // IDX_BATCH)
    def _(g):
        pltpu.sync_copy(src_hbm.at[pl.ds(cbase + g * IDX_BATCH, IDX_BATCH)], idx_s)
        pltpu.sync_copy(dst_hbm.at[pl.ds(cbase + g * IDX_BATCH, IDX_BATCH)], idx_d)

        @pl.loop(0, IDX_BATCH)
        def _(j):
            pltpu.async_copy(h_hbm.at[idx_s.at[j]], rows, sem).wait()
            pltpu.sync_copy(rows, acc_sh.at[idx_d.at[j]], add=True)

    plsc.subcore_barrier()

    off = 0
    for sz in SLABS:
        pltpu.sync_copy(acc_sh.at[pl.ds(rbase + off, sz)],
                        rows.at[pl.ds(0, sz)])
        pltpu.sync_copy(rows.at[pl.ds(0, sz)],
                        out_hbm.at[cid, pl.ds(rbase + off, sz)])
        off += sz


def _sc_aggregate(h, src2d, dst2d):
    mesh = plsc.VectorSubcoreMesh(core_axis_name="c", subcore_axis_name="s")
    call = pl.kernel(
        _agg_body,
        out_type=jax.ShapeDtypeStruct((2, NP, HID), jnp.float32),
        mesh=mesh,
        scratch_types=[
            pltpu.VMEM_SHARED((NP, HID), jnp.float32),
            pltpu.VMEM((IDX_BATCH, CHUNK), jnp.int32),
            pltpu.VMEM((IDX_BATCH, CHUNK), jnp.int32),
            pltpu.VMEM((CHUNK, HID), jnp.float32),
            pltpu.SemaphoreType.DMA,
        ],
    )
    return call(h, src2d, dst2d)


def _deg_body(dst_hbm, out_hbm, ldeg, idx_d):
    cid = lax.axis_index("c")
    sid = lax.axis_index("s")
    wid = cid * 16 + sid

    @pl.loop(0, NP // 16)
    def _(i):
        ldeg[pl.ds(i * 16, 16)] = jnp.zeros((16,), jnp.float32)

    cbase = wid * CHUNKS_PER_WORKER
    ones16 = jnp.ones((16,), jnp.float32)

    @pl.loop(0, CHUNKS_PER_WORKER // IDX_BATCH)
    def _(g):
        pltpu.sync_copy(dst_hbm.at[pl.ds(cbase + g * IDX_BATCH, IDX_BATCH)], idx_d)

        @pl.loop(0, IDX_BATCH)
        def _(j):
            for k in range(CHUNK // 16):
                iv = idx_d[j, pl.ds(k * 16, 16)]
                plsc.addupdate_scatter(ldeg, [iv], ones16)

    pltpu.sync_copy(ldeg, out_hbm.at[pl.ds(wid * NP, NP)])


def _no_layout_params():
    cp = pltpu.CompilerParams()
    if "needs_layout_passes" in pltpu.CompilerParams.__dataclass_fields__:
        cp = dataclasses.replace(cp, needs_layout_passes=False)
    return cp


def _sc_degree(dst2d):
    mesh = plsc.VectorSubcoreMesh(core_axis_name="c", subcore_axis_name="s")
    call = pl.kernel(
        _deg_body,
        out_type=jax.ShapeDtypeStruct((32 * NP,), jnp.float32),
        mesh=mesh,
        compiler_params=_no_layout_params(),
        scratch_types=[
            pltpu.VMEM((NP,), jnp.float32),
            pltpu.VMEM((IDX_BATCH, CHUNK), jnp.int32),
        ],
    )
    return call(dst2d)


def _tgt_gather_body(h2_hbm, x_hbm, tgt_hbm, emb_hbm, xg_hbm, tidx, buf, sem):
    cid = lax.axis_index("c")
    sid = lax.axis_index("s")
    wid = sid * 2 + cid
    pltpu.sync_copy(tgt_hbm, tidx)
    pltpu.async_copy(h2_hbm.at[tidx.at[wid]], buf, sem).wait()
    pltpu.sync_copy(buf, emb_hbm.at[pl.ds(wid * 32, 32)])
    pltpu.async_copy(x_hbm.at[tidx.at[wid]], buf, sem).wait()
    pltpu.sync_copy(buf, xg_hbm.at[pl.ds(wid * 32, 32)])


def _sc_tgt_gather(h2, x, tgt2d):
    mesh = plsc.VectorSubcoreMesh(core_axis_name="c", subcore_axis_name="s")
    call = pl.kernel(
        _tgt_gather_body,
        out_type=(jax.ShapeDtypeStruct((B, HID), jnp.float32),
                  jax.ShapeDtypeStruct((B, HID), jnp.float32)),
        mesh=mesh,
        scratch_types=[
            pltpu.VMEM((32, 32), jnp.int32),
            pltpu.VMEM((32, HID), jnp.float32),
            pltpu.SemaphoreType.DMA,
        ],
    )
    return call(h2, x, tgt2d)


def kernel(tgt_id, node_feat, x, edge_index, W_mlp, b_mlp, bn_gamma, bn_beta,
           W_self1, W_neigh1, b1, W_self2, W_neigh2, b2, W_pred, b_pred):
    f32 = jnp.float32
    tgt2d = tgt_id.astype(jnp.int32).reshape(32, 32)
    src = edge_index[0].astype(jnp.int32)
    dst = edge_index[1].astype(jnp.int32)
    pad_i = jnp.arange(EP - E, dtype=jnp.int32)
    src2d = jnp.concatenate([src, pad_i % N]).reshape(NCHUNKS, CHUNK)
    dst2d = jnp.concatenate([dst, pad_i % (NP - N) + N]).reshape(NCHUNKS, CHUNK)

    b_mlp2 = b_mlp.reshape(1, HID)
    g2 = bn_gamma.reshape(1, HID)
    be2 = bn_beta.reshape(1, HID)
    b1_2 = b1.reshape(1, HID)
    b2_2 = b2.reshape(1, HID)
    bp2 = b_pred.reshape(1, HID)

    deg_t = _sc_degree(dst2d).reshape(32, NP).T

    h0 = _tc_call(_mlp_bn_body, jax.ShapeDtypeStruct((N, HID), f32))(
        node_feat, W_mlp, b_mlp2, g2, be2)

    acc1 = _sc_aggregate(h0, src2d, dst2d)
    h1 = _tc_call(_sage_body, jax.ShapeDtypeStruct((N, HID), f32))(
        h0, acc1, deg_t, W_self1, W_neigh1, b1_2)

    acc2 = _sc_aggregate(h1, src2d, dst2d)
    h2 = _tc_call(_sage_body, jax.ShapeDtypeStruct((N, HID), f32))(
        h1, acc2, deg_t, W_self2, W_neigh2, b2_2)

    embed, xg = _sc_tgt_gather(h2, x, tgt2d)

    x_prime, loss_arr = _tc_call(
        _pred_body,
        (jax.ShapeDtypeStruct((B, HID), f32),
         jax.ShapeDtypeStruct((1, 1), f32)))(embed, xg, W_pred, bp2)

    return (loss_arr[0, 0], x_prime, embed)


# speedup vs baseline: 11.3457x; 1.4035x over previous
import dataclasses
import functools

import jax
import jax.numpy as jnp
from jax import lax
from jax.experimental import pallas as pl
from jax.experimental.pallas import tpu as pltpu
from jax.experimental.pallas import tpu_sc as plsc

N = 10000
E = 320000
HID = 128
B = 1024

NP = 10112
ROWS_PER_TILE = NP // 16
CHUNK = 128
EP = 327680
NCHUNKS = EP // CHUNK
CHUNKS_PER_WORKER = NCHUNKS // 32
IDX_BATCH = 16
SLABS = (128, 128, 128, 128, ROWS_PER_TILE - 4 * 128)


def _leaky(v):
    return jnp.where(v >= 0, v, 0.1 * v)


def _mlp_bn_body(nf, w, b, g, be, out):
    h = jnp.dot(nf[...], w[...], preferred_element_type=jnp.float32) + b[...]
    h = _leaky(h)
    mu = jnp.mean(h, axis=0, keepdims=True)
    var = jnp.mean((h - mu) ** 2, axis=0, keepdims=True)
    out[...] = (h - mu) * lax.rsqrt(var + 1e-5) * g[...] + be[...]


def _sage_body(hin, accp, deg_t, ws, wn, b, out):
    acc = accp[0] + accp[1]
    deg = jnp.sum(deg_t[...], axis=1, keepdims=True)
    inv = 1.0 / jnp.maximum(deg, 1.0)
    aggm = (acc * inv)[:N]
    out[...] = _leaky(
        jnp.dot(hin[...], ws[...], preferred_element_type=jnp.float32)
        + jnp.dot(aggm, wn[...], preferred_element_type=jnp.float32)
        + b[...])


def _pred_body(embed, xg, wp, bp, xp_out, loss_out):
    e = _leaky(embed[...])
    xp = jnp.dot(e, wp[...], preferred_element_type=jnp.float32) + bp[...]
    xp_out[...] = xp
    d = xp - xg[...]
    loss_out[...] = jnp.mean(d * d).reshape(1, 1)


def _tc_call(body, out_shapes):
    return pl.pallas_call(body, out_shape=out_shapes)


def _agg_body(h_hbm, src_hbm, dst_hbm, out_hbm, acc_sh, idx_s, idx_d,
              rows, rows2, sem, sem2):
    cid = lax.axis_index("c")
    sid = lax.axis_index("s")
    rbase = sid * ROWS_PER_TILE

    @pl.loop(0, CHUNK)
    def _(r):
        for k in range(HID // 16):
            rows[r, pl.ds(k * 16, 16)] = jnp.zeros((16,), jnp.float32)

    off = 0
    for sz in SLABS:
        pltpu.sync_copy(rows.at[pl.ds(0, sz)],
                        acc_sh.at[pl.ds(rbase + off, sz)])
        off += sz

    plsc.subcore_barrier()

    cbase = (cid * 16 + sid) * CHUNKS_PER_WORKER

    @pl.loop(0, CHUNKS_PER_WORKER // IDX_BATCH)
    def _(g):
        pltpu.sync_copy(src_hbm.at[pl.ds(cbase + g * IDX_BATCH, IDX_BATCH)], idx_s)
        pltpu.sync_copy(dst_hbm.at[pl.ds(cbase + g * IDX_BATCH, IDX_BATCH)], idx_d)

        pltpu.async_copy(h_hbm.at[idx_s.at[0]], rows, sem)

        @pl.loop(0, IDX_BATCH // 2 - 1)
        def _(p):
            pltpu.async_copy(h_hbm.at[idx_s.at[2 * p + 1]], rows2, sem2)
            pltpu.make_async_copy(h_hbm.at[idx_s.at[2 * p]], rows, sem).wait()
            pltpu.sync_copy(rows, acc_sh.at[idx_d.at[2 * p]], add=True)
            pltpu.async_copy(h_hbm.at[idx_s.at[2 * p + 2]], rows, sem)
            pltpu.make_async_copy(h_hbm.at[idx_s.at[2 * p + 1]], rows2, sem2).wait()
            pltpu.sync_copy(rows2, acc_sh.at[idx_d.at[2 * p + 1]], add=True)

        pltpu.async_copy(h_hbm.at[idx_s.at[IDX_BATCH - 1]], rows2, sem2)
        pltpu.make_async_copy(h_hbm.at[idx_s.at[IDX_BATCH - 2]], rows, sem).wait()
        pltpu.sync_copy(rows, acc_sh.at[idx_d.at[IDX_BATCH - 2]], add=True)
        pltpu.make_async_copy(h_hbm.at[idx_s.at[IDX_BATCH - 1]], rows2, sem2).wait()
        pltpu.sync_copy(rows2, acc_sh.at[idx_d.at[IDX_BATCH - 1]], add=True)

    plsc.subcore_barrier()

    off = 0
    for sz in SLABS:
        pltpu.sync_copy(acc_sh.at[pl.ds(rbase + off, sz)],
                        rows.at[pl.ds(0, sz)])
        pltpu.sync_copy(rows.at[pl.ds(0, sz)],
                        out_hbm.at[cid, pl.ds(rbase + off, sz)])
        off += sz


def _sc_aggregate(h, src2d, dst2d):
    mesh = plsc.VectorSubcoreMesh(core_axis_name="c", subcore_axis_name="s")
    call = pl.kernel(
        _agg_body,
        out_type=jax.ShapeDtypeStruct((2, NP, HID), jnp.float32),
        mesh=mesh,
        scratch_types=[
            pltpu.VMEM_SHARED((NP, HID), jnp.float32),
            pltpu.VMEM((IDX_BATCH, CHUNK), jnp.int32),
            pltpu.VMEM((IDX_BATCH, CHUNK), jnp.int32),
            pltpu.VMEM((CHUNK, HID), jnp.float32),
            pltpu.VMEM((CHUNK, HID), jnp.float32),
            pltpu.SemaphoreType.DMA,
            pltpu.SemaphoreType.DMA,
        ],
    )
    return call(h, src2d, dst2d)


def _deg_body(dst_hbm, out_hbm, ldeg, idx_d):
    cid = lax.axis_index("c")
    sid = lax.axis_index("s")
    wid = cid * 16 + sid

    @pl.loop(0, NP // 16)
    def _(i):
        ldeg[pl.ds(i * 16, 16)] = jnp.zeros((16,), jnp.float32)

    cbase = wid * CHUNKS_PER_WORKER
    ones16 = jnp.ones((16,), jnp.float32)

    @pl.loop(0, CHUNKS_PER_WORKER // IDX_BATCH)
    def _(g):
        pltpu.sync_copy(dst_hbm.at[pl.ds(cbase + g * IDX_BATCH, IDX_BATCH)], idx_d)

        @pl.loop(0, IDX_BATCH)
        def _(j):
            for k in range(CHUNK // 16):
                iv = idx_d[j, pl.ds(k * 16, 16)]
                plsc.addupdate_scatter(ldeg, [iv], ones16)

    pltpu.sync_copy(ldeg, out_hbm.at[pl.ds(wid * NP, NP)])


def _no_layout_params():
    cp = pltpu.CompilerParams()
    if "needs_layout_passes" in pltpu.CompilerParams.__dataclass_fields__:
        cp = dataclasses.replace(cp, needs_layout_passes=False)
    return cp


def _sc_degree(dst2d):
    mesh = plsc.VectorSubcoreMesh(core_axis_name="c", subcore_axis_name="s")
    call = pl.kernel(
        _deg_body,
        out_type=jax.ShapeDtypeStruct((32 * NP,), jnp.float32),
        mesh=mesh,
        compiler_params=_no_layout_params(),
        scratch_types=[
            pltpu.VMEM((NP,), jnp.float32),
            pltpu.VMEM((IDX_BATCH, CHUNK), jnp.int32),
        ],
    )
    return call(dst2d)


def _tgt_gather_body(h2_hbm, x_hbm, tgt_hbm, emb_hbm, xg_hbm, tidx, buf, sem):
    cid = lax.axis_index("c")
    sid = lax.axis_index("s")
    wid = sid * 2 + cid
    pltpu.sync_copy(tgt_hbm, tidx)
    pltpu.async_copy(h2_hbm.at[tidx.at[wid]], buf, sem).wait()
    pltpu.sync_copy(buf, emb_hbm.at[pl.ds(wid * 32, 32)])
    pltpu.async_copy(x_hbm.at[tidx.at[wid]], buf, sem).wait()
    pltpu.sync_copy(buf, xg_hbm.at[pl.ds(wid * 32, 32)])


def _sc_tgt_gather(h2, x, tgt2d):
    mesh = plsc.VectorSubcoreMesh(core_axis_name="c", subcore_axis_name="s")
    call = pl.kernel(
        _tgt_gather_body,
        out_type=(jax.ShapeDtypeStruct((B, HID), jnp.float32),
                  jax.ShapeDtypeStruct((B, HID), jnp.float32)),
        mesh=mesh,
        scratch_types=[
            pltpu.VMEM((32, 32), jnp.int32),
            pltpu.VMEM((32, HID), jnp.float32),
            pltpu.SemaphoreType.DMA,
        ],
    )
    return call(h2, x, tgt2d)


def kernel(tgt_id, node_feat, x, edge_index, W_mlp, b_mlp, bn_gamma, bn_beta,
           W_self1, W_neigh1, b1, W_self2, W_neigh2, b2, W_pred, b_pred):
    f32 = jnp.float32
    tgt2d = tgt_id.astype(jnp.int32).reshape(32, 32)
    src = edge_index[0].astype(jnp.int32)
    dst = edge_index[1].astype(jnp.int32)
    pad_i = jnp.arange(EP - E, dtype=jnp.int32)
    src2d = jnp.concatenate([src, pad_i % N]).reshape(NCHUNKS, CHUNK)
    dst2d = jnp.concatenate([dst, pad_i % (NP - N) + N]).reshape(NCHUNKS, CHUNK)

    b_mlp2 = b_mlp.reshape(1, HID)
    g2 = bn_gamma.reshape(1, HID)
    be2 = bn_beta.reshape(1, HID)
    b1_2 = b1.reshape(1, HID)
    b2_2 = b2.reshape(1, HID)
    bp2 = b_pred.reshape(1, HID)

    deg_t = _sc_degree(dst2d).reshape(32, NP).T

    h0 = _tc_call(_mlp_bn_body, jax.ShapeDtypeStruct((N, HID), f32))(
        node_feat, W_mlp, b_mlp2, g2, be2)

    acc1 = _sc_aggregate(h0, src2d, dst2d)
    h1 = _tc_call(_sage_body, jax.ShapeDtypeStruct((N, HID), f32))(
        h0, acc1, deg_t, W_self1, W_neigh1, b1_2)

    acc2 = _sc_aggregate(h1, src2d, dst2d)
    h2 = _tc_call(_sage_body, jax.ShapeDtypeStruct((N, HID), f32))(
        h1, acc2, deg_t, W_self2, W_neigh2, b2_2)

    embed, xg = _sc_tgt_gather(h2, x, tgt2d)

    x_prime, loss_arr = _tc_call(
        _pred_body,
        (jax.ShapeDtypeStruct((B, HID), f32),
         jax.ShapeDtypeStruct((1, 1), f32)))(embed, xg, W_pred, bp2)

    return (loss_arr[0, 0], x_prime, embed)


# speedup vs baseline: 12.0460x; 1.0617x over previous
import dataclasses
import functools

import jax
import jax.numpy as jnp
from jax import lax
from jax.experimental import pallas as pl
from jax.experimental.pallas import tpu as pltpu
from jax.experimental.pallas import tpu_sc as plsc

N = 10000
E = 320000
HID = 128
B = 1024

NP = 10112
ROWS_PER_TILE = NP // 16
CHUNK = 128
EP = 327680
NCHUNKS = EP // CHUNK
CHUNKS_PER_WORKER = NCHUNKS // 32
IDX_BATCH = 40
SLABS = (128, 128, 128, 128, ROWS_PER_TILE - 4 * 128)


def _leaky(v):
    return jnp.where(v >= 0, v, 0.1 * v)


def _mlp_bn_body(nf, w, b, g, be, out):
    h = jnp.dot(nf[...], w[...], preferred_element_type=jnp.float32) + b[...]
    h = _leaky(h)
    mu = jnp.mean(h, axis=0, keepdims=True)
    var = jnp.mean((h - mu) ** 2, axis=0, keepdims=True)
    out[...] = (h - mu) * lax.rsqrt(var + 1e-5) * g[...] + be[...]


def _sage_body(hin, accp, deg_t, ws, wn, b, out):
    acc = accp[0] + accp[1]
    deg = jnp.sum(deg_t[...], axis=1, keepdims=True)
    inv = 1.0 / jnp.maximum(deg, 1.0)
    aggm = (acc * inv)[:N]
    out[...] = _leaky(
        jnp.dot(hin[...], ws[...], preferred_element_type=jnp.float32)
        + jnp.dot(aggm, wn[...], preferred_element_type=jnp.float32)
        + b[...])


def _pred_body(embed, xg, wp, bp, xp_out, loss_out):
    e = _leaky(embed[...])
    xp = jnp.dot(e, wp[...], preferred_element_type=jnp.float32) + bp[...]
    xp_out[...] = xp
    d = xp - xg[...]
    loss_out[...] = jnp.mean(d * d).reshape(1, 1)


def _tc_call(body, out_shapes):
    return pl.pallas_call(body, out_shape=out_shapes)


def _agg_body(h_hbm, src_hbm, dst_hbm, out_hbm, acc_sh, idx_s, idx_d,
              rows, rows2, sem, sem2):
    cid = lax.axis_index("c")
    sid = lax.axis_index("s")
    rbase = sid * ROWS_PER_TILE

    @pl.loop(0, CHUNK)
    def _(r):
        for k in range(HID // 16):
            rows[r, pl.ds(k * 16, 16)] = jnp.zeros((16,), jnp.float32)

    off = 0
    for sz in SLABS:
        pltpu.sync_copy(rows.at[pl.ds(0, sz)],
                        acc_sh.at[pl.ds(rbase + off, sz)])
        off += sz

    plsc.subcore_barrier()

    cbase = (cid * 16 + sid) * CHUNKS_PER_WORKER

    @pl.loop(0, CHUNKS_PER_WORKER // IDX_BATCH)
    def _(g):
        pltpu.sync_copy(src_hbm.at[pl.ds(cbase + g * IDX_BATCH, IDX_BATCH)], idx_s)
        pltpu.sync_copy(dst_hbm.at[pl.ds(cbase + g * IDX_BATCH, IDX_BATCH)], idx_d)

        pltpu.async_copy(h_hbm.at[idx_s.at[0]], rows, sem)

        @pl.loop(0, IDX_BATCH // 2 - 1)
        def _(p):
            pltpu.async_copy(h_hbm.at[idx_s.at[2 * p + 1]], rows2, sem2)
            pltpu.make_async_copy(h_hbm.at[idx_s.at[2 * p]], rows, sem).wait()
            pltpu.sync_copy(rows, acc_sh.at[idx_d.at[2 * p]], add=True)
            pltpu.async_copy(h_hbm.at[idx_s.at[2 * p + 2]], rows, sem)
            pltpu.make_async_copy(h_hbm.at[idx_s.at[2 * p + 1]], rows2, sem2).wait()
            pltpu.sync_copy(rows2, acc_sh.at[idx_d.at[2 * p + 1]], add=True)

        pltpu.async_copy(h_hbm.at[idx_s.at[IDX_BATCH - 1]], rows2, sem2)
        pltpu.make_async_copy(h_hbm.at[idx_s.at[IDX_BATCH - 2]], rows, sem).wait()
        pltpu.sync_copy(rows, acc_sh.at[idx_d.at[IDX_BATCH - 2]], add=True)
        pltpu.make_async_copy(h_hbm.at[idx_s.at[IDX_BATCH - 1]], rows2, sem2).wait()
        pltpu.sync_copy(rows2, acc_sh.at[idx_d.at[IDX_BATCH - 1]], add=True)

    plsc.subcore_barrier()

    off = 0
    for sz in SLABS:
        pltpu.sync_copy(acc_sh.at[pl.ds(rbase + off, sz)],
                        rows.at[pl.ds(0, sz)])
        pltpu.sync_copy(rows.at[pl.ds(0, sz)],
                        out_hbm.at[cid, pl.ds(rbase + off, sz)])
        off += sz


def _sc_aggregate(h, src2d, dst2d):
    mesh = plsc.VectorSubcoreMesh(core_axis_name="c", subcore_axis_name="s")
    call = pl.kernel(
        _agg_body,
        out_type=jax.ShapeDtypeStruct((2, NP, HID), jnp.float32),
        mesh=mesh,
        scratch_types=[
            pltpu.VMEM_SHARED((NP, HID), jnp.float32),
            pltpu.VMEM((IDX_BATCH, CHUNK), jnp.int32),
            pltpu.VMEM((IDX_BATCH, CHUNK), jnp.int32),
            pltpu.VMEM((CHUNK, HID), jnp.float32),
            pltpu.VMEM((CHUNK, HID), jnp.float32),
            pltpu.SemaphoreType.DMA,
            pltpu.SemaphoreType.DMA,
        ],
    )
    return call(h, src2d, dst2d)


def _deg_body(dst_hbm, out_hbm, ldeg, idx_d):
    cid = lax.axis_index("c")
    sid = lax.axis_index("s")
    wid = cid * 16 + sid

    @pl.loop(0, NP // 16)
    def _(i):
        ldeg[pl.ds(i * 16, 16)] = jnp.zeros((16,), jnp.float32)

    cbase = wid * CHUNKS_PER_WORKER
    ones16 = jnp.ones((16,), jnp.float32)

    @pl.loop(0, CHUNKS_PER_WORKER // IDX_BATCH)
    def _(g):
        pltpu.sync_copy(dst_hbm.at[pl.ds(cbase + g * IDX_BATCH, IDX_BATCH)], idx_d)

        @pl.loop(0, IDX_BATCH)
        def _(j):
            for k in range(CHUNK // 16):
                iv = idx_d[j, pl.ds(k * 16, 16)]
                plsc.addupdate_scatter(ldeg, [iv], ones16)

    pltpu.sync_copy(ldeg, out_hbm.at[pl.ds(wid * NP, NP)])


def _no_layout_params():
    cp = pltpu.CompilerParams()
    if "needs_layout_passes" in pltpu.CompilerParams.__dataclass_fields__:
        cp = dataclasses.replace(cp, needs_layout_passes=False)
    return cp


def _sc_degree(dst2d):
    mesh = plsc.VectorSubcoreMesh(core_axis_name="c", subcore_axis_name="s")
    call = pl.kernel(
        _deg_body,
        out_type=jax.ShapeDtypeStruct((32 * NP,), jnp.float32),
        mesh=mesh,
        compiler_params=_no_layout_params(),
        scratch_types=[
            pltpu.VMEM((NP,), jnp.float32),
            pltpu.VMEM((IDX_BATCH, CHUNK), jnp.int32),
        ],
    )
    return call(dst2d)


def _tgt_gather_body(h2_hbm, x_hbm, tgt_hbm, emb_hbm, xg_hbm, tidx, buf, sem):
    cid = lax.axis_index("c")
    sid = lax.axis_index("s")
    wid = sid * 2 + cid
    pltpu.sync_copy(tgt_hbm, tidx)
    pltpu.async_copy(h2_hbm.at[tidx.at[wid]], buf, sem).wait()
    pltpu.sync_copy(buf, emb_hbm.at[pl.ds(wid * 32, 32)])
    pltpu.async_copy(x_hbm.at[tidx.at[wid]], buf, sem).wait()
    pltpu.sync_copy(buf, xg_hbm.at[pl.ds(wid * 32, 32)])


def _sc_tgt_gather(h2, x, tgt2d):
    mesh = plsc.VectorSubcoreMesh(core_axis_name="c", subcore_axis_name="s")
    call = pl.kernel(
        _tgt_gather_body,
        out_type=(jax.ShapeDtypeStruct((B, HID), jnp.float32),
                  jax.ShapeDtypeStruct((B, HID), jnp.float32)),
        mesh=mesh,
        scratch_types=[
            pltpu.VMEM((32, 32), jnp.int32),
            pltpu.VMEM((32, HID), jnp.float32),
            pltpu.SemaphoreType.DMA,
        ],
    )
    return call(h2, x, tgt2d)


def kernel(tgt_id, node_feat, x, edge_index, W_mlp, b_mlp, bn_gamma, bn_beta,
           W_self1, W_neigh1, b1, W_self2, W_neigh2, b2, W_pred, b_pred):
    f32 = jnp.float32
    tgt2d = tgt_id.astype(jnp.int32).reshape(32, 32)
    src = edge_index[0].astype(jnp.int32)
    dst = edge_index[1].astype(jnp.int32)
    pad_i = jnp.arange(EP - E, dtype=jnp.int32)
    src2d = jnp.concatenate([src, pad_i % N]).reshape(NCHUNKS, CHUNK)
    dst2d = jnp.concatenate([dst, pad_i % (NP - N) + N]).reshape(NCHUNKS, CHUNK)

    b_mlp2 = b_mlp.reshape(1, HID)
    g2 = bn_gamma.reshape(1, HID)
    be2 = bn_beta.reshape(1, HID)
    b1_2 = b1.reshape(1, HID)
    b2_2 = b2.reshape(1, HID)
    bp2 = b_pred.reshape(1, HID)

    deg_t = _sc_degree(dst2d).reshape(32, NP).T

    h0 = _tc_call(_mlp_bn_body, jax.ShapeDtypeStruct((N, HID), f32))(
        node_feat, W_mlp, b_mlp2, g2, be2)

    acc1 = _sc_aggregate(h0, src2d, dst2d)
    h1 = _tc_call(_sage_body, jax.ShapeDtypeStruct((N, HID), f32))(
        h0, acc1, deg_t, W_self1, W_neigh1, b1_2)

    acc2 = _sc_aggregate(h1, src2d, dst2d)
    h2 = _tc_call(_sage_body, jax.ShapeDtypeStruct((N, HID), f32))(
        h1, acc2, deg_t, W_self2, W_neigh2, b2_2)

    embed, xg = _sc_tgt_gather(h2, x, tgt2d)

    x_prime, loss_arr = _tc_call(
        _pred_body,
        (jax.ShapeDtypeStruct((B, HID), f32),
         jax.ShapeDtypeStruct((1, 1), f32)))(embed, xg, W_pred, bp2)

    return (loss_arr[0, 0], x_prime, embed)
